# Initial kernel scaffold; baseline (speedup 1.0000x reference)
import functools

import numpy as np
import jax
import jax.numpy as jnp
from jax.experimental import pallas as pl
from jax.experimental.pallas import tpu as pltpu

CUTOFF = 10.0
NUM_LAYERS = 6
HID = 128
NG = 50
OUTC = 32
NUM_CLASS = 2
NUM_GRAPHS = 64
LOG2 = float(np.log(2.0))

TJ = 128
TI = 128
JC = 8

_F32 = jnp.float32


def _ssp(x):
    return jnp.maximum(x, 0.0) + jnp.log1p(jnp.exp(-jnp.abs(x))) - LOG2


def _lrelu(x):
    return jnp.where(x >= 0, x, 0.01 * x)


def _dot(a, b, prec=jax.lax.Precision.HIGHEST):
    return jax.lax.dot_general(a, b, (((a.ndim - 1,), (0,)), ((), ())),
                               precision=prec,
                               preferred_element_type=_F32)


def _schnet_body(tlo_ref, thi_ref,
                 pos_ref, z_ref, b_ref, bT_ref, emb_ref,
                 m1t_ref, m1b_ref, m2t_ref, m2b_ref, lint_ref,
                 v1t_ref, v1b_ref, v2t_ref, v2b_ref,
                 u1t_ref, u1b_ref, u2t_ref, u2b_ref,
                 fw1t_ref, fb1_ref, fa_ref, fw2t_ref, fb2_ref,
                 off_ref, coeff_ref,
                 out_ref,
                 v_scr, agg_scr):
    m = pl.program_id(0)
    ph = pl.program_id(1)
    bj = pl.program_id(2)
    j0 = bj * TJ

    @pl.when(ph == 0)
    def _init():
        z_t = z_ref[pl.ds(j0, TJ), :]
        oh = (z_t == jax.lax.broadcasted_iota(jnp.int32, (TJ, 100), 1))
        v_scr[0, pl.ds(j0, TJ), :] = _dot(oh.astype(_F32), emb_ref[...])

        @pl.when(bj == 0)
        def _():
            out_ref[...] = jnp.zeros_like(out_ref)

    @pl.when((ph >= 1) & (ph <= NUM_LAYERS))
    def _layer():
        l = ph - 1
        rp = jax.lax.rem(l, 2)
        wp = 1 - rp
        agg_scr[...] = jnp.zeros_like(agg_scr)
        b_j = b_ref[pl.ds(j0, TJ), :]
        pos_j = pos_ref[pl.ds(j0, TJ), :]
        offs = off_ref[...]
        coeff = coeff_ref[...]
        m1t = m1t_ref[...]
        m1b = m1b_ref[...]
        m2t = m2t_ref[...]
        m2b = m2b_ref[...]
        lint = lint_ref[...]

        def bi_body(bi, carry):
            i0 = bi * TI
            v_i = v_scr[rp, pl.ds(i0, TI), :]
            vl3 = _dot(v_i, lint).reshape(1, TI, HID)
            pos_i = pos_ref[pl.ds(i0, TI), :]
            pxi = pos_i[:, 0:1].reshape(1, TI, 1)
            pyi = pos_i[:, 1:2].reshape(1, TI, 1)
            pzi = pos_i[:, 2:3].reshape(1, TI, 1)
            sqi = pxi * pxi + pyi * pyi + pzi * pzi
            bi3 = b_ref[pl.ds(i0, TI), :].reshape(1, TI, 1)
            ig = i0 + jax.lax.broadcasted_iota(jnp.int32, (JC, TI, 1), 1)
            for jc in range(0, TJ, JC):
                pc = pos_j[jc:jc + JC, :]
                pxj = pc[:, 0:1].reshape(JC, 1, 1)
                pyj = pc[:, 1:2].reshape(JC, 1, 1)
                pzj = pc[:, 2:3].reshape(JC, 1, 1)
                sqj = pxj * pxj + pyj * pyj + pzj * pzj
                bj3 = b_j[jc:jc + JC, :].reshape(JC, 1, 1)
                jg = (j0 + jc) + jax.lax.broadcasted_iota(
                    jnp.int32, (JC, TI, 1), 0)
                dx = pxi - pxj
                dy = pyi - pyj
                dz = pzi - pzj
                d2 = dx * dx + dy * dy + dz * dz
                d2q = sqi + sqj - 2.0 * (pxi * pxj + pyi * pyj + pzi * pzj)
                mask = ((d2q <= CUTOFF * CUTOFF) & (bi3 == bj3) & (ig != jg))
                dist = jnp.sqrt(d2)
                cc = 0.5 * (jnp.cos(dist * jnp.pi / CUTOFF) + 1.0)
                pm = jnp.where(mask, cc, 0.0)
                dflat = dist.reshape(JC * TI, 1)
                demb = jnp.exp(coeff * (dflat - offs) ** 2)
                a = _ssp(_dot(demb, m1t) + m1b)
                w = _dot(a, m2t) + m2b
                e = w.reshape(JC, TI, HID) * pm * vl3
                agg_scr[pl.ds(jc, JC), :] += jnp.sum(e, axis=1)
            return carry

        jax.lax.fori_loop(tlo_ref[m, bj], thi_ref[m, bj], bi_body, 0)

        agg = agg_scr[...]
        h = _ssp(_dot(agg, v1t_ref[...]) + v1b_ref[...])
        upd = _dot(h, v2t_ref[...]) + v2b_ref[...]
        v_scr[wp, pl.ds(j0, TJ), :] = v_scr[rp, pl.ds(j0, TJ), :] + upd

    @pl.when(ph == NUM_LAYERS + 1)
    def _readout():
        rp = NUM_LAYERS % 2
        v = v_scr[rp, pl.ds(j0, TJ), :]
        h = _ssp(_dot(v, u1t_ref[...]) + u1b_ref[...])
        u = _dot(h, u2t_ref[...]) + u2b_ref[...]
        bT = bT_ref[:, pl.ds(j0, TJ)]
        oh = (jax.lax.broadcasted_iota(jnp.int32, (NUM_GRAPHS, TJ), 0) == bT)
        out_ref[...] += _dot(oh.astype(_F32), u)

    @pl.when((ph == NUM_LAYERS + 2) & (bj == 0))
    def _fc():
        x = out_ref[...]
        h = _dot(x, fw1t_ref[...]) + fb1_ref[...]
        h = jnp.where(h >= 0, h, fa_ref[...] * h)
        out_ref[...] = _dot(h, fw2t_ref[...]) + fb2_ref[...]


def _cnn_body(x0_ref, w1_ref, b1_ref, w2_ref, b2_ref,
              w31_ref, b31_ref, w32_ref, b32_ref,
              w4_ref, b4_ref, fc1_ref, fc1b_ref, fc2_ref, fc2b_ref,
              out_ref):
    rows = x0_ref.shape[0]
    rh = 32
    hidx = jax.lax.rem(jax.lax.broadcasted_iota(jnp.int32, (rows, 1), 0), rh)

    def conv(x, w_ref, b_ref):
        c = x.shape[1]
        zr = jnp.zeros((1, c), _F32)
        xm = jnp.concatenate([zr, x[:-1, :]], axis=0)
        xm = jnp.where(hidx == 0, 0.0, xm)
        xp = jnp.concatenate([x[1:, :], zr], axis=0)
        xp = jnp.where(hidx == rh - 1, 0.0, xp)
        return (_dot(xm, w_ref[0]) + _dot(x, w_ref[1]) + _dot(xp, w_ref[2])
                + b_ref[...])

    x = _lrelu(conv(x0_ref[...], w1_ref, b1_ref))
    x = _lrelu(conv(x, w2_ref, b2_ref))
    res = x
    x = _lrelu(conv(x, w31_ref, b31_ref))
    x = _lrelu(conv(x, w32_ref, b32_ref))
    x = res + x
    x = _lrelu(conv(x, w4_ref, b4_ref))
    x3 = x.reshape(rows // rh, rh, 256)
    acc = jnp.zeros((rows // rh, 64), _F32)
    for h in range(rh):
        acc = acc + _dot(x3[:, h, :], fc1_ref[h])
    acc = _lrelu(acc + fc1b_ref[...])
    out_ref[...] = _dot(acc, fc2_ref[...]) + fc2b_ref[...]


def _tile_bounds(batch, ntj):
    idx = jnp.arange(ntj) * TJ
    starts = batch[idx]
    ends = batch[idx + (TJ - 1)]
    i_lo = jnp.searchsorted(batch, starts, side="left")
    i_hi = jnp.searchsorted(batch, ends, side="right")
    t_lo = (i_lo // TI).astype(jnp.int32)
    t_hi = ((i_hi + TI - 1) // TI).astype(jnp.int32)
    return t_lo, t_hi


def _stack_schnet(params):
    ms = [params["m1"], params["m2"]]
    out = {}
    out["emb"] = jnp.stack([p["emb"] for p in ms])
    for nm, src in [("m1t", "mlp1_w"), ("m2t", "mlp2_w"), ("lint", "lin_w"),
                    ("v1t", "v1_w"), ("v2t", "v2_w")]:
        out[nm] = jnp.stack([
            jnp.stack([lp[src].T for lp in p["layers"]]) for p in ms])
    for nm, src in [("m1b", "mlp1_b"), ("m2b", "mlp2_b"),
                    ("v1b", "v1_b"), ("v2b", "v2_b")]:
        out[nm] = jnp.stack([
            jnp.stack([lp[src][None, :] for lp in p["layers"]]) for p in ms])
    out["u1t"] = jnp.stack([p["u1_w"].T for p in ms])
    out["u1b"] = jnp.stack([p["u1_b"][None, :] for p in ms])
    out["u2t"] = jnp.stack([p["u2_w"].T for p in ms])
    out["u2b"] = jnp.stack([p["u2_b"][None, :] for p in ms])
    fcs = [params["fc1"], params["fc2"]]
    out["fw1t"] = jnp.stack([p["w1"].T for p in fcs])
    out["fb1"] = jnp.stack([p["b1"][None, :] for p in fcs])
    out["fa"] = jnp.stack([p["a"][None, :] for p in fcs])
    out["fw2t"] = jnp.stack([p["w2"].T for p in fcs])
    out["fb2"] = jnp.stack([p["b2"][None, :] for p in fcs])
    return out


@jax.jit
def kernel(pos1, z1, pos1_batch, pos2, z2, pos2_batch, params):
    n = pos1.shape[0]
    ntj = n // TJ
    g = NUM_GRAPHS

    sp = _stack_schnet(params)
    pos = jnp.stack([pos1, pos2])
    z = jnp.stack([z1, z2]).astype(jnp.int32)[:, :, None]
    b1 = pos1_batch.astype(jnp.int32)
    b2 = pos2_batch.astype(jnp.int32)
    bat = jnp.stack([b1, b2])[:, :, None]
    batT = jnp.stack([b1, b2])[:, None, :]
    offs = jnp.linspace(0.0, CUTOFF, NG)
    coeff = (-0.5 / (offs[1] - offs[0]) ** 2)[None, None]
    offs = offs[None, :]

    tlo1, thi1 = _tile_bounds(b1, ntj)
    tlo2, thi2 = _tile_bounds(b2, ntj)
    tlo = jnp.stack([tlo1, tlo2])
    thi = jnp.stack([thi1, thi2])

    def full(shape):
        nd = len(shape)
        return pl.BlockSpec(shape, lambda m, ph, bj, *_: (0,) * nd)

    def per_m(shape):
        nd = len(shape)
        return pl.BlockSpec((None,) + shape,
                            lambda m, ph, bj, *_: (m,) + (0,) * nd)

    def per_ml(shape):
        nd = len(shape)

        def imap(m, ph, bj, *_):
            l = jnp.clip(ph - 1, 0, NUM_LAYERS - 1)
            return (m, l) + (0,) * nd

        return pl.BlockSpec((None, None) + shape, imap)

    grid_spec = pltpu.PrefetchScalarGridSpec(
        num_scalar_prefetch=2,
        grid=(2, NUM_LAYERS + 3, ntj),
        in_specs=[
            per_m((n, 3)),
            per_m((n, 1)),
            per_m((n, 1)),
            per_m((1, n)),
            per_m((100, HID)),
            per_ml((NG, HID)),
            per_ml((1, HID)),
            per_ml((HID, HID)),
            per_ml((1, HID)),
            per_ml((HID, HID)),
            per_ml((HID, HID)),
            per_ml((1, HID)),
            per_ml((HID, HID)),
            per_ml((1, HID)),
            per_m((HID, HID // 2)),
            per_m((1, HID // 2)),
            per_m((HID // 2, OUTC)),
            per_m((1, OUTC)),
            per_m((OUTC, 64)),
            per_m((1, 64)),
            per_m((1, 1)),
            per_m((64, OUTC)),
            per_m((1, OUTC)),
            full((1, NG)),
            full((1, 1)),
        ],
        out_specs=per_m((g, OUTC)),
        scratch_shapes=[
            pltpu.VMEM((2, n, HID), _F32),
            pltpu.VMEM((TJ, HID), _F32),
        ],
    )

    pred = pl.pallas_call(
        _schnet_body,
        grid_spec=grid_spec,
        out_shape=jax.ShapeDtypeStruct((2, g, OUTC), _F32),
    )(tlo, thi,
      pos, z, bat, batT, sp["emb"],
      sp["m1t"], sp["m1b"], sp["m2t"], sp["m2b"], sp["lint"],
      sp["v1t"], sp["v1b"], sp["v2t"], sp["v2b"],
      sp["u1t"], sp["u1b"], sp["u2t"], sp["u2b"],
      sp["fw1t"], sp["fb1"], sp["fa"], sp["fw2t"], sp["fb2"],
      offs, coeff)

    x0 = jnp.concatenate([pred[0].reshape(-1, 1), pred[1].reshape(-1, 1)],
                         axis=1)

    cp = params["cnn"]
    w1 = jnp.transpose(cp["c1_w"], (2, 1, 0))
    w2 = jnp.transpose(cp["c2_w"], (2, 1, 0))
    w31 = jnp.transpose(cp["c31_w"], (2, 1, 0))
    w32 = jnp.transpose(cp["c32_w"], (2, 1, 0))
    w4 = jnp.transpose(cp["c4_w"], (2, 1, 0))
    fc1 = jnp.transpose(cp["fc1_w"].reshape(64, 256, 32), (2, 1, 0))
    out = pl.pallas_call(
        _cnn_body,
        out_shape=jax.ShapeDtypeStruct((g, NUM_CLASS), _F32),
    )(x0, w1, cp["c1_b"][None, :], w2, cp["c2_b"][None, :],
      w31, cp["c31_b"][None, :], w32, cp["c32_b"][None, :],
      w4, cp["c4_b"][None, :], fc1, cp["fc1_b"][None, :],
      cp["fc2_w"].T, cp["fc2_b"][None, :])
    return out


# speedup vs baseline: 10.3779x; 10.3779x over previous
import functools

import numpy as np
import jax
import jax.numpy as jnp
from jax.experimental import pallas as pl
from jax.experimental.pallas import tpu as pltpu

CUTOFF = 10.0
NUM_LAYERS = 6
HID = 128
NG = 50
OUTC = 32
NUM_CLASS = 2
NUM_GRAPHS = 64
LOG2 = float(np.log(2.0))

TJ = 128
TI = 128
JC = 8

_F32 = jnp.float32


def _ssp(x):
    return jnp.maximum(x, 0.0) + jnp.log1p(jnp.exp(-jnp.abs(x))) - LOG2


def _lrelu(x):
    return jnp.where(x >= 0, x, 0.01 * x)


def _dot(a, b, prec=jax.lax.Precision.HIGHEST):
    return jax.lax.dot_general(a, b, (((a.ndim - 1,), (0,)), ((), ())),
                               precision=prec,
                               preferred_element_type=_F32)


def _dotd(a, b):
    return jax.lax.dot_general(a.astype(jnp.bfloat16), b.astype(jnp.bfloat16),
                               (((a.ndim - 1,), (0,)), ((), ())),
                               preferred_element_type=_F32)


def _schnet_body(tlo_ref, thi_ref,
                 pos_ref, z_ref, b_ref, bT_ref, emb_ref,
                 m1t_ref, m1b_ref, m2t_ref, m2b_ref, lint_ref,
                 v1t_ref, v1b_ref, v2t_ref, v2b_ref,
                 u1t_ref, u1b_ref, u2t_ref, u2b_ref,
                 fw1t_ref, fb1_ref, fa_ref, fw2t_ref, fb2_ref,
                 off_ref, coeff_ref,
                 out_ref,
                 v_scr, agg_scr):
    m = pl.program_id(0)
    ph = pl.program_id(1)
    bj = pl.program_id(2)
    j0 = bj * TJ

    @pl.when(ph == 0)
    def _init():
        z_t = z_ref[pl.ds(j0, TJ), :]
        oh = (z_t == jax.lax.broadcasted_iota(jnp.int32, (TJ, 100), 1))
        v_scr[0, pl.ds(j0, TJ), :] = _dot(oh.astype(_F32), emb_ref[...])

        @pl.when(bj == 0)
        def _():
            out_ref[...] = jnp.zeros_like(out_ref)

    @pl.when((ph >= 1) & (ph <= NUM_LAYERS))
    def _layer():
        l = ph - 1
        rp = jax.lax.rem(l, 2)
        wp = 1 - rp
        agg_scr[...] = jnp.zeros_like(agg_scr)
        b_j = b_ref[pl.ds(j0, TJ), :]
        pos_j = pos_ref[pl.ds(j0, TJ), :]
        offs = off_ref[...]
        coeff = coeff_ref[...]
        m1t = m1t_ref[...]
        m1b = m1b_ref[...]
        m2t = m2t_ref[...]
        m2b = m2b_ref[...]
        lint = lint_ref[...]

        def bi_body(bi, carry):
            i0 = bi * TI
            v_i = v_scr[rp, pl.ds(i0, TI), :]
            vl3 = _dotd(v_i, lint).reshape(1, TI, HID)
            pos_i = pos_ref[pl.ds(i0, TI), :]
            pxi = pos_i[:, 0:1].reshape(1, TI, 1)
            pyi = pos_i[:, 1:2].reshape(1, TI, 1)
            pzi = pos_i[:, 2:3].reshape(1, TI, 1)
            sqi = pxi * pxi + pyi * pyi + pzi * pzi
            bi3 = b_ref[pl.ds(i0, TI), :].reshape(1, TI, 1)
            ig = i0 + jax.lax.broadcasted_iota(jnp.int32, (JC, TI, 1), 1)
            for jc in range(0, TJ, JC):
                pc = pos_j[jc:jc + JC, :]
                pxj = pc[:, 0:1].reshape(JC, 1, 1)
                pyj = pc[:, 1:2].reshape(JC, 1, 1)
                pzj = pc[:, 2:3].reshape(JC, 1, 1)
                sqj = pxj * pxj + pyj * pyj + pzj * pzj
                bj3 = b_j[jc:jc + JC, :].reshape(JC, 1, 1)
                jg = (j0 + jc) + jax.lax.broadcasted_iota(
                    jnp.int32, (JC, TI, 1), 0)
                dx = pxi - pxj
                dy = pyi - pyj
                dz = pzi - pzj
                d2 = dx * dx + dy * dy + dz * dz
                d2q = sqi + sqj - 2.0 * (pxi * pxj + pyi * pyj + pzi * pzj)
                mask = ((d2q <= CUTOFF * CUTOFF) & (bi3 == bj3) & (ig != jg))
                dist = jnp.sqrt(d2)
                cc = 0.5 * (jnp.cos(dist * jnp.pi / CUTOFF) + 1.0)
                pm = jnp.where(mask, cc, 0.0)
                dflat = dist.reshape(JC * TI, 1)
                demb = jnp.exp(coeff * (dflat - offs) ** 2)
                a = _ssp(_dotd(demb, m1t) + m1b)
                w = _dotd(a, m2t) + m2b
                e = w.reshape(JC, TI, HID) * pm * vl3
                agg_scr[pl.ds(jc, JC), :] += jnp.sum(e, axis=1)
            return carry

        jax.lax.fori_loop(tlo_ref[m, bj], thi_ref[m, bj], bi_body, 0)

        agg = agg_scr[...]
        h = _ssp(_dotd(agg, v1t_ref[...]) + v1b_ref[...])
        upd = _dotd(h, v2t_ref[...]) + v2b_ref[...]
        v_scr[wp, pl.ds(j0, TJ), :] = v_scr[rp, pl.ds(j0, TJ), :] + upd

    @pl.when(ph == NUM_LAYERS + 1)
    def _readout():
        rp = NUM_LAYERS % 2
        v = v_scr[rp, pl.ds(j0, TJ), :]
        h = _ssp(_dotd(v, u1t_ref[...]) + u1b_ref[...])
        u = _dotd(h, u2t_ref[...]) + u2b_ref[...]
        bT = bT_ref[:, pl.ds(j0, TJ)]
        oh = (jax.lax.broadcasted_iota(jnp.int32, (NUM_GRAPHS, TJ), 0) == bT)
        out_ref[...] += _dot(oh.astype(_F32), u)

    @pl.when((ph == NUM_LAYERS + 2) & (bj == 0))
    def _fc():
        x = out_ref[...]
        h = _dotd(x, fw1t_ref[...]) + fb1_ref[...]
        h = jnp.where(h >= 0, h, fa_ref[...] * h)
        out_ref[...] = _dotd(h, fw2t_ref[...]) + fb2_ref[...]


def _cnn_body(x0_ref, w1_ref, b1_ref, w2_ref, b2_ref,
              w31_ref, b31_ref, w32_ref, b32_ref,
              w4_ref, b4_ref, fc1_ref, fc1b_ref, fc2_ref, fc2b_ref,
              out_ref):
    rows = x0_ref.shape[0]
    rh = 32
    hidx = jax.lax.rem(jax.lax.broadcasted_iota(jnp.int32, (rows, 1), 0), rh)

    def conv(x, w_ref, b_ref):
        c = x.shape[1]
        zr = jnp.zeros((1, c), _F32)
        xm = jnp.concatenate([zr, x[:-1, :]], axis=0)
        xm = jnp.where(hidx == 0, 0.0, xm)
        xp = jnp.concatenate([x[1:, :], zr], axis=0)
        xp = jnp.where(hidx == rh - 1, 0.0, xp)
        return (_dotd(xm, w_ref[0]) + _dotd(x, w_ref[1]) + _dotd(xp, w_ref[2])
                + b_ref[...])

    x = _lrelu(conv(x0_ref[...], w1_ref, b1_ref))
    x = _lrelu(conv(x, w2_ref, b2_ref))
    res = x
    x = _lrelu(conv(x, w31_ref, b31_ref))
    x = _lrelu(conv(x, w32_ref, b32_ref))
    x = res + x
    x = _lrelu(conv(x, w4_ref, b4_ref))
    x3 = x.reshape(rows // rh, rh, 256)
    acc = jnp.zeros((rows // rh, 64), _F32)
    for h in range(rh):
        acc = acc + _dotd(x3[:, h, :], fc1_ref[h])
    acc = _lrelu(acc + fc1b_ref[...])
    out_ref[...] = _dotd(acc, fc2_ref[...]) + fc2b_ref[...]


def _tile_bounds(batch, ntj):
    idx = jnp.arange(ntj) * TJ
    starts = batch[idx]
    ends = batch[idx + (TJ - 1)]
    i_lo = jnp.searchsorted(batch, starts, side="left")
    i_hi = jnp.searchsorted(batch, ends, side="right")
    t_lo = (i_lo // TI).astype(jnp.int32)
    t_hi = ((i_hi + TI - 1) // TI).astype(jnp.int32)
    return t_lo, t_hi


def _stack_schnet(params):
    ms = [params["m1"], params["m2"]]
    out = {}
    out["emb"] = jnp.stack([p["emb"] for p in ms])
    for nm, src in [("m1t", "mlp1_w"), ("m2t", "mlp2_w"), ("lint", "lin_w"),
                    ("v1t", "v1_w"), ("v2t", "v2_w")]:
        out[nm] = jnp.stack([
            jnp.stack([lp[src].T for lp in p["layers"]]) for p in ms])
    for nm, src in [("m1b", "mlp1_b"), ("m2b", "mlp2_b"),
                    ("v1b", "v1_b"), ("v2b", "v2_b")]:
        out[nm] = jnp.stack([
            jnp.stack([lp[src][None, :] for lp in p["layers"]]) for p in ms])
    out["u1t"] = jnp.stack([p["u1_w"].T for p in ms])
    out["u1b"] = jnp.stack([p["u1_b"][None, :] for p in ms])
    out["u2t"] = jnp.stack([p["u2_w"].T for p in ms])
    out["u2b"] = jnp.stack([p["u2_b"][None, :] for p in ms])
    fcs = [params["fc1"], params["fc2"]]
    out["fw1t"] = jnp.stack([p["w1"].T for p in fcs])
    out["fb1"] = jnp.stack([p["b1"][None, :] for p in fcs])
    out["fa"] = jnp.stack([p["a"][None, :] for p in fcs])
    out["fw2t"] = jnp.stack([p["w2"].T for p in fcs])
    out["fb2"] = jnp.stack([p["b2"][None, :] for p in fcs])
    return out


@jax.jit
def kernel(pos1, z1, pos1_batch, pos2, z2, pos2_batch, params):
    n = pos1.shape[0]
    ntj = n // TJ
    g = NUM_GRAPHS

    sp = _stack_schnet(params)
    pos = jnp.stack([pos1, pos2])
    z = jnp.stack([z1, z2]).astype(jnp.int32)[:, :, None]
    b1 = pos1_batch.astype(jnp.int32)
    b2 = pos2_batch.astype(jnp.int32)
    bat = jnp.stack([b1, b2])[:, :, None]
    batT = jnp.stack([b1, b2])[:, None, :]
    offs = jnp.linspace(0.0, CUTOFF, NG)
    coeff = (-0.5 / (offs[1] - offs[0]) ** 2)[None, None]
    offs = offs[None, :]

    tlo1, thi1 = _tile_bounds(b1, ntj)
    tlo2, thi2 = _tile_bounds(b2, ntj)
    tlo = jnp.stack([tlo1, tlo2])
    thi = jnp.stack([thi1, thi2])

    def full(shape):
        nd = len(shape)
        return pl.BlockSpec(shape, lambda m, ph, bj, *_: (0,) * nd)

    def per_m(shape):
        nd = len(shape)
        return pl.BlockSpec((None,) + shape,
                            lambda m, ph, bj, *_: (m,) + (0,) * nd)

    def per_ml(shape):
        nd = len(shape)

        def imap(m, ph, bj, *_):
            l = jnp.clip(ph - 1, 0, NUM_LAYERS - 1)
            return (m, l) + (0,) * nd

        return pl.BlockSpec((None, None) + shape, imap)

    grid_spec = pltpu.PrefetchScalarGridSpec(
        num_scalar_prefetch=2,
        grid=(2, NUM_LAYERS + 3, ntj),
        in_specs=[
            per_m((n, 3)),
            per_m((n, 1)),
            per_m((n, 1)),
            per_m((1, n)),
            per_m((100, HID)),
            per_ml((NG, HID)),
            per_ml((1, HID)),
            per_ml((HID, HID)),
            per_ml((1, HID)),
            per_ml((HID, HID)),
            per_ml((HID, HID)),
            per_ml((1, HID)),
            per_ml((HID, HID)),
            per_ml((1, HID)),
            per_m((HID, HID // 2)),
            per_m((1, HID // 2)),
            per_m((HID // 2, OUTC)),
            per_m((1, OUTC)),
            per_m((OUTC, 64)),
            per_m((1, 64)),
            per_m((1, 1)),
            per_m((64, OUTC)),
            per_m((1, OUTC)),
            full((1, NG)),
            full((1, 1)),
        ],
        out_specs=per_m((g, OUTC)),
        scratch_shapes=[
            pltpu.VMEM((2, n, HID), _F32),
            pltpu.VMEM((TJ, HID), _F32),
        ],
    )

    pred = pl.pallas_call(
        _schnet_body,
        grid_spec=grid_spec,
        out_shape=jax.ShapeDtypeStruct((2, g, OUTC), _F32),
    )(tlo, thi,
      pos, z, bat, batT, sp["emb"],
      sp["m1t"], sp["m1b"], sp["m2t"], sp["m2b"], sp["lint"],
      sp["v1t"], sp["v1b"], sp["v2t"], sp["v2b"],
      sp["u1t"], sp["u1b"], sp["u2t"], sp["u2b"],
      sp["fw1t"], sp["fb1"], sp["fa"], sp["fw2t"], sp["fb2"],
      offs, coeff)

    x0 = jnp.concatenate([pred[0].reshape(-1, 1), pred[1].reshape(-1, 1)],
                         axis=1)

    cp = params["cnn"]
    w1 = jnp.transpose(cp["c1_w"], (2, 1, 0))
    w2 = jnp.transpose(cp["c2_w"], (2, 1, 0))
    w31 = jnp.transpose(cp["c31_w"], (2, 1, 0))
    w32 = jnp.transpose(cp["c32_w"], (2, 1, 0))
    w4 = jnp.transpose(cp["c4_w"], (2, 1, 0))
    fc1 = jnp.transpose(cp["fc1_w"].reshape(64, 256, 32), (2, 1, 0))
    out = pl.pallas_call(
        _cnn_body,
        out_shape=jax.ShapeDtypeStruct((g, NUM_CLASS), _F32),
    )(x0, w1, cp["c1_b"][None, :], w2, cp["c2_b"][None, :],
      w31, cp["c31_b"][None, :], w32, cp["c32_b"][None, :],
      w4, cp["c4_b"][None, :], fc1, cp["fc1_b"][None, :],
      cp["fc2_w"].T, cp["fc2_b"][None, :])
    return out


# speedup vs baseline: 92.9088x; 8.9526x over previous
import functools

import numpy as np
import jax
import jax.numpy as jnp
from jax.experimental import pallas as pl
from jax.experimental.pallas import tpu as pltpu

CUTOFF = 10.0
NUM_LAYERS = 6
HID = 128
NG = 50
OUTC = 32
NUM_CLASS = 2
NUM_GRAPHS = 64
LOG2 = float(np.log(2.0))

TJ = 128
JC = 8
WI = 64

_F32 = jnp.float32


def _ssp(x):
    return jnp.maximum(x, 0.0) + jnp.log1p(jnp.exp(-jnp.abs(x))) - LOG2


def _lrelu(x):
    return jnp.where(x >= 0, x, 0.01 * x)


def _dot(a, b, prec=jax.lax.Precision.HIGHEST):
    return jax.lax.dot_general(a, b, (((a.ndim - 1,), (0,)), ((), ())),
                               precision=prec,
                               preferred_element_type=_F32)


def _dotd(a, b):
    return jax.lax.dot_general(a.astype(jnp.bfloat16), b.astype(jnp.bfloat16),
                               (((a.ndim - 1,), (0,)), ((), ())),
                               preferred_element_type=_F32)


def _schnet_body(ilo_ref, wn_ref,
                 pos_ref, z_ref, b_ref, bT_ref, emb_ref,
                 m1t_ref, m1b_ref, m2t_ref, m2b_ref, lint_ref,
                 v1t_ref, v1b_ref, v2t_ref, v2b_ref,
                 u1t_ref, u1b_ref, u2t_ref, u2b_ref,
                 fw1t_ref, fb1_ref, fa_ref, fw2t_ref, fb2_ref,
                 off_ref, coeff_ref,
                 out_ref,
                 v_scr, vl_scr, agg_scr):
    m = pl.program_id(0)
    ph = pl.program_id(1)
    bj = pl.program_id(2)
    j0 = bj * TJ
    n = v_scr.shape[1]
    nti = n // TJ
    nmax = n - WI
    nch_tile = TJ // JC

    @pl.when(ph == 0)
    def _init():
        z_t = z_ref[pl.ds(j0, TJ), :]
        oh = (z_t == jax.lax.broadcasted_iota(jnp.int32, (TJ, 100), 1))
        v_scr[0, pl.ds(j0, TJ), :] = _dot(oh.astype(_F32), emb_ref[...])

        @pl.when(bj == 0)
        def _():
            out_ref[...] = jnp.zeros_like(out_ref)

    @pl.when((ph >= 1) & (ph <= NUM_LAYERS))
    def _layer():
        l = ph - 1
        rp = jax.lax.rem(l, 2)
        wp = 1 - rp
        lint = lint_ref[...]

        @pl.when(bj == 0)
        def _vl():
            for t in range(nti):
                v_t = v_scr[rp, pl.ds(t * TJ, TJ), :]
                vl_scr[pl.ds(t * TJ, TJ), :] = _dotd(v_t, lint)

        agg_scr[...] = jnp.zeros_like(agg_scr)
        offs = off_ref[...]
        coeff = coeff_ref[...]
        m1t = m1t_ref[...]
        m1b = m1b_ref[...]
        m2t = m2t_ref[...]
        m2b = m2b_ref[...]
        lane = jax.lax.broadcasted_iota(jnp.int32, (JC, JC * WI), 1)
        sub = jax.lax.broadcasted_iota(jnp.int32, (JC, JC * WI), 0)
        selb = (lane // WI) == sub

        def chunk_body(c, carry):
            jcd = j0 + c * JC
            cj = bj * nch_tile + c
            pj = pos_ref[pl.ds(jcd, JC), :]
            pxj = pj[:, 0:1]
            pyj = pj[:, 1:2]
            pzj = pj[:, 2:3]
            sqj = pxj * pxj + pyj * pyj + pzj * pzj
            bjc = b_ref[pl.ds(jcd, JC), :]
            jg2 = jcd + jax.lax.broadcasted_iota(jnp.int32, (JC, WI), 0)
            pxj3 = pxj.reshape(JC, 1, 1)
            pyj3 = pyj.reshape(JC, 1, 1)
            pzj3 = pzj.reshape(JC, 1, 1)
            i_base = ilo_ref[m, cj]

            def win_body(t, carry2):
                i0o = i_base + t * WI
                i0c = jnp.minimum(i0o, nmax)
                pw = pos_ref[pl.ds(i0c, WI), :]
                pwT = jnp.transpose(pw)
                pxi = pwT[0:1, :]
                pyi = pwT[1:2, :]
                pzi = pwT[2:3, :]
                bic = jnp.transpose(b_ref[pl.ds(i0c, WI), :])
                ig2 = i0c + jax.lax.broadcasted_iota(jnp.int32, (JC, WI), 1)
                sqi = pxi * pxi + pyi * pyi + pzi * pzi
                d2q = sqj + sqi - 2.0 * (pxj * pxi + pyj * pyi + pzj * pzi)
                mask = ((d2q <= CUTOFF * CUTOFF) & (bjc == bic)
                        & (jg2 != ig2) & (ig2 >= i0o))
                dx = pxj - pxi
                dy = pyj - pyi
                dz = pzj - pzi
                distd = jnp.sqrt(dx * dx + dy * dy + dz * dz)
                ccd = 0.5 * (jnp.cos(distd * jnp.pi / CUTOFF) + 1.0)
                pmd = jnp.where(mask, ccd, 0.0)
                pmblk = jnp.where(selb,
                                  jnp.concatenate([pmd] * JC, axis=1), 0.0)
                dx3 = pxj3 - pw[:, 0:1].reshape(1, WI, 1)
                dy3 = pyj3 - pw[:, 1:2].reshape(1, WI, 1)
                dz3 = pzj3 - pw[:, 2:3].reshape(1, WI, 1)
                dist3 = jnp.sqrt(dx3 * dx3 + dy3 * dy3 + dz3 * dz3)
                dflat = dist3.reshape(JC * WI, 1)
                demb = jnp.exp(coeff * (dflat - offs) ** 2)
                aa = _ssp(_dotd(demb, m1t) + m1b)
                w = _dotd(aa, m2t) + m2b
                vlw = vl_scr[pl.ds(i0c, WI), :].reshape(1, WI, HID)
                e0 = (w.reshape(JC, WI, HID) * vlw).reshape(JC * WI, HID)
                agg_scr[pl.ds(c * JC, JC), :] += _dot(pmblk, e0)
                return carry2

            jax.lax.fori_loop(0, wn_ref[m, cj], win_body, 0)
            return carry

        jax.lax.fori_loop(0, nch_tile, chunk_body, 0)

        agg = agg_scr[...]
        h = _ssp(_dotd(agg, v1t_ref[...]) + v1b_ref[...])
        upd = _dotd(h, v2t_ref[...]) + v2b_ref[...]
        v_scr[wp, pl.ds(j0, TJ), :] = v_scr[rp, pl.ds(j0, TJ), :] + upd

    @pl.when(ph == NUM_LAYERS + 1)
    def _readout():
        rp = NUM_LAYERS % 2
        v = v_scr[rp, pl.ds(j0, TJ), :]
        h = _ssp(_dotd(v, u1t_ref[...]) + u1b_ref[...])
        u = _dotd(h, u2t_ref[...]) + u2b_ref[...]
        bT = bT_ref[:, pl.ds(j0, TJ)]
        oh = (jax.lax.broadcasted_iota(jnp.int32, (NUM_GRAPHS, TJ), 0) == bT)
        out_ref[...] += _dot(oh.astype(_F32), u)

    @pl.when((ph == NUM_LAYERS + 2) & (bj == 0))
    def _fc():
        x = out_ref[...]
        h = _dotd(x, fw1t_ref[...]) + fb1_ref[...]
        h = jnp.where(h >= 0, h, fa_ref[...] * h)
        out_ref[...] = _dotd(h, fw2t_ref[...]) + fb2_ref[...]


def _cnn_body(x0_ref, w1_ref, b1_ref, w2_ref, b2_ref,
              w31_ref, b31_ref, w32_ref, b32_ref,
              w4_ref, b4_ref, fc1_ref, fc1b_ref, fc2_ref, fc2b_ref,
              out_ref):
    rows = x0_ref.shape[0]
    rh = 32
    hidx = jax.lax.rem(jax.lax.broadcasted_iota(jnp.int32, (rows, 1), 0), rh)

    def conv(x, w_ref, b_ref):
        c = x.shape[1]
        zr = jnp.zeros((1, c), _F32)
        xm = jnp.concatenate([zr, x[:-1, :]], axis=0)
        xm = jnp.where(hidx == 0, 0.0, xm)
        xp = jnp.concatenate([x[1:, :], zr], axis=0)
        xp = jnp.where(hidx == rh - 1, 0.0, xp)
        return (_dotd(xm, w_ref[0]) + _dotd(x, w_ref[1]) + _dotd(xp, w_ref[2])
                + b_ref[...])

    x = _lrelu(conv(x0_ref[...], w1_ref, b1_ref))
    x = _lrelu(conv(x, w2_ref, b2_ref))
    res = x
    x = _lrelu(conv(x, w31_ref, b31_ref))
    x = _lrelu(conv(x, w32_ref, b32_ref))
    x = res + x
    x = _lrelu(conv(x, w4_ref, b4_ref))
    x3 = x.reshape(rows // rh, rh, 256)
    acc = jnp.zeros((rows // rh, 64), _F32)
    for h in range(rh):
        acc = acc + _dotd(x3[:, h, :], fc1_ref[h])
    acc = _lrelu(acc + fc1b_ref[...])
    out_ref[...] = _dotd(acc, fc2_ref[...]) + fc2b_ref[...]


def _chunk_windows(batch):
    nch = batch.shape[0] // JC
    cidx = jnp.arange(nch) * JC
    lo = jnp.searchsorted(batch, batch[cidx], side="left")
    hi = jnp.searchsorted(batch, batch[cidx + (JC - 1)], side="right")
    lo8 = ((lo // 8) * 8).astype(jnp.int32)
    wn = ((hi - lo8 + (WI - 1)) // WI).astype(jnp.int32)
    return lo8, wn


def _stack_schnet(params):
    ms = [params["m1"], params["m2"]]
    out = {}
    out["emb"] = jnp.stack([p["emb"] for p in ms])
    for nm, src in [("m1t", "mlp1_w"), ("m2t", "mlp2_w"), ("lint", "lin_w"),
                    ("v1t", "v1_w"), ("v2t", "v2_w")]:
        out[nm] = jnp.stack([
            jnp.stack([lp[src].T for lp in p["layers"]]) for p in ms])
    for nm, src in [("m1b", "mlp1_b"), ("m2b", "mlp2_b"),
                    ("v1b", "v1_b"), ("v2b", "v2_b")]:
        out[nm] = jnp.stack([
            jnp.stack([lp[src][None, :] for lp in p["layers"]]) for p in ms])
    out["u1t"] = jnp.stack([p["u1_w"].T for p in ms])
    out["u1b"] = jnp.stack([p["u1_b"][None, :] for p in ms])
    out["u2t"] = jnp.stack([p["u2_w"].T for p in ms])
    out["u2b"] = jnp.stack([p["u2_b"][None, :] for p in ms])
    fcs = [params["fc1"], params["fc2"]]
    out["fw1t"] = jnp.stack([p["w1"].T for p in fcs])
    out["fb1"] = jnp.stack([p["b1"][None, :] for p in fcs])
    out["fa"] = jnp.stack([p["a"][None, :] for p in fcs])
    out["fw2t"] = jnp.stack([p["w2"].T for p in fcs])
    out["fb2"] = jnp.stack([p["b2"][None, :] for p in fcs])
    return out


@jax.jit
def kernel(pos1, z1, pos1_batch, pos2, z2, pos2_batch, params):
    n = pos1.shape[0]
    ntj = n // TJ
    g = NUM_GRAPHS

    sp = _stack_schnet(params)
    pos = jnp.stack([pos1, pos2])
    z = jnp.stack([z1, z2]).astype(jnp.int32)[:, :, None]
    b1 = pos1_batch.astype(jnp.int32)
    b2 = pos2_batch.astype(jnp.int32)
    bat = jnp.stack([b1, b2])[:, :, None]
    batT = jnp.stack([b1, b2])[:, None, :]
    offs = jnp.linspace(0.0, CUTOFF, NG)
    coeff = (-0.5 / (offs[1] - offs[0]) ** 2)[None, None]
    offs = offs[None, :]

    ilo1, wn1 = _chunk_windows(b1)
    ilo2, wn2 = _chunk_windows(b2)
    ilo = jnp.stack([ilo1, ilo2])
    wn = jnp.stack([wn1, wn2])

    def full(shape):
        nd = len(shape)
        return pl.BlockSpec(shape, lambda m, ph, bj, *_: (0,) * nd)

    def per_m(shape):
        nd = len(shape)
        return pl.BlockSpec((None,) + shape,
                            lambda m, ph, bj, *_: (m,) + (0,) * nd)

    def per_ml(shape):
        nd = len(shape)

        def imap(m, ph, bj, *_):
            l = jnp.clip(ph - 1, 0, NUM_LAYERS - 1)
            return (m, l) + (0,) * nd

        return pl.BlockSpec((None, None) + shape, imap)

    grid_spec = pltpu.PrefetchScalarGridSpec(
        num_scalar_prefetch=2,
        grid=(2, NUM_LAYERS + 3, ntj),
        in_specs=[
            per_m((n, 3)),
            per_m((n, 1)),
            per_m((n, 1)),
            per_m((1, n)),
            per_m((100, HID)),
            per_ml((NG, HID)),
            per_ml((1, HID)),
            per_ml((HID, HID)),
            per_ml((1, HID)),
            per_ml((HID, HID)),
            per_ml((HID, HID)),
            per_ml((1, HID)),
            per_ml((HID, HID)),
            per_ml((1, HID)),
            per_m((HID, HID // 2)),
            per_m((1, HID // 2)),
            per_m((HID // 2, OUTC)),
            per_m((1, OUTC)),
            per_m((OUTC, 64)),
            per_m((1, 64)),
            per_m((1, 1)),
            per_m((64, OUTC)),
            per_m((1, OUTC)),
            full((1, NG)),
            full((1, 1)),
        ],
        out_specs=per_m((g, OUTC)),
        scratch_shapes=[
            pltpu.VMEM((2, n, HID), _F32),
            pltpu.VMEM((n, HID), _F32),
            pltpu.VMEM((TJ, HID), _F32),
        ],
    )

    pred = pl.pallas_call(
        _schnet_body,
        grid_spec=grid_spec,
        out_shape=jax.ShapeDtypeStruct((2, g, OUTC), _F32),
    )(ilo, wn,
      pos, z, bat, batT, sp["emb"],
      sp["m1t"], sp["m1b"], sp["m2t"], sp["m2b"], sp["lint"],
      sp["v1t"], sp["v1b"], sp["v2t"], sp["v2b"],
      sp["u1t"], sp["u1b"], sp["u2t"], sp["u2b"],
      sp["fw1t"], sp["fb1"], sp["fa"], sp["fw2t"], sp["fb2"],
      offs, coeff)

    x0 = jnp.concatenate([pred[0].reshape(-1, 1), pred[1].reshape(-1, 1)],
                         axis=1)

    cp = params["cnn"]
    w1 = jnp.transpose(cp["c1_w"], (2, 1, 0))
    w2 = jnp.transpose(cp["c2_w"], (2, 1, 0))
    w31 = jnp.transpose(cp["c31_w"], (2, 1, 0))
    w32 = jnp.transpose(cp["c32_w"], (2, 1, 0))
    w4 = jnp.transpose(cp["c4_w"], (2, 1, 0))
    fc1 = jnp.transpose(cp["fc1_w"].reshape(64, 256, 32), (2, 1, 0))
    out = pl.pallas_call(
        _cnn_body,
        out_shape=jax.ShapeDtypeStruct((g, NUM_CLASS), _F32),
    )(x0, w1, cp["c1_b"][None, :], w2, cp["c2_b"][None, :],
      w31, cp["c31_b"][None, :], w32, cp["c32_b"][None, :],
      w4, cp["c4_b"][None, :], fc1, cp["fc1_b"][None, :],
      cp["fc2_w"].T, cp["fc2_b"][None, :])
    return out


# speedup vs baseline: 93.2222x; 1.0034x over previous
import functools

import numpy as np
import jax
import jax.numpy as jnp
from jax.experimental import pallas as pl
from jax.experimental.pallas import tpu as pltpu
from jax.experimental.pallas import tpu_sc as plsc

CUTOFF = 10.0
NUM_LAYERS = 6
HID = 128
NG = 50
OUTC = 32
NUM_CLASS = 2
NUM_GRAPHS = 64
LOG2 = float(np.log(2.0))

TJ = 128
JC = 8
WI = 64

_F32 = jnp.float32


def _ssp(x):
    return jnp.maximum(x, 0.0) + jnp.log1p(jnp.exp(-jnp.abs(x))) - LOG2


def _lrelu(x):
    return jnp.where(x >= 0, x, 0.01 * x)


def _dot(a, b, prec=jax.lax.Precision.HIGHEST):
    return jax.lax.dot_general(a, b, (((a.ndim - 1,), (0,)), ((), ())),
                               precision=prec,
                               preferred_element_type=_F32)


def _dotd(a, b):
    return jax.lax.dot_general(a.astype(jnp.bfloat16), b.astype(jnp.bfloat16),
                               (((a.ndim - 1,), (0,)), ((), ())),
                               preferred_element_type=_F32)


def _schnet_body(ilo_ref, wn_ref,
                 pos_ref, z_ref, b_ref, bT_ref, emb_ref,
                 m1t_ref, m1b_ref, m2t_ref, m2b_ref, lint_ref,
                 v1t_ref, v1b_ref, v2t_ref, v2b_ref,
                 u1t_ref, u1b_ref, u2t_ref, u2b_ref,
                 off_ref, coeff_ref,
                 out_ref,
                 v_scr, vl_scr, agg_scr):
    m = pl.program_id(0)
    ph = pl.program_id(1)
    bj = pl.program_id(2)
    j0 = bj * TJ
    n = v_scr.shape[1]
    nti = n // TJ
    nmax = n - WI
    nch_tile = TJ // JC

    @pl.when(ph == 0)
    def _init():
        z_t = z_ref[pl.ds(j0, TJ), :]
        oh = (z_t == jax.lax.broadcasted_iota(jnp.int32, (TJ, 100), 1))
        v_scr[0, pl.ds(j0, TJ), :] = _dot(oh.astype(_F32), emb_ref[...])

    @pl.when((ph >= 1) & (ph <= NUM_LAYERS))
    def _layer():
        l = ph - 1
        rp = jax.lax.rem(l, 2)
        wp = 1 - rp
        lint = lint_ref[...]

        @pl.when(bj == 0)
        def _vl():
            for t in range(nti):
                v_t = v_scr[rp, pl.ds(t * TJ, TJ), :]
                vl_scr[pl.ds(t * TJ, TJ), :] = _dotd(v_t, lint)

        agg_scr[...] = jnp.zeros_like(agg_scr)
        offs = off_ref[...]
        coeff = coeff_ref[...]
        m1t = m1t_ref[...]
        m1b = m1b_ref[...]
        m2t = m2t_ref[...]
        m2b = m2b_ref[...]
        lane = jax.lax.broadcasted_iota(jnp.int32, (JC, JC * WI), 1)
        sub = jax.lax.broadcasted_iota(jnp.int32, (JC, JC * WI), 0)
        selb = (lane // WI) == sub

        def chunk_body(c, carry):
            jcd = j0 + c * JC
            cj = bj * nch_tile + c
            pj = pos_ref[pl.ds(jcd, JC), :]
            pxj = pj[:, 0:1]
            pyj = pj[:, 1:2]
            pzj = pj[:, 2:3]
            sqj = pxj * pxj + pyj * pyj + pzj * pzj
            bjc = b_ref[pl.ds(jcd, JC), :]
            jg2 = jcd + jax.lax.broadcasted_iota(jnp.int32, (JC, WI), 0)
            pxj3 = pxj.reshape(JC, 1, 1)
            pyj3 = pyj.reshape(JC, 1, 1)
            pzj3 = pzj.reshape(JC, 1, 1)
            i_base = ilo_ref[m, cj]

            def win_body(t, carry2):
                i0o = i_base + t * WI
                i0c = jnp.minimum(i0o, nmax)
                pw = pos_ref[pl.ds(i0c, WI), :]
                pwT = jnp.transpose(pw)
                pxi = pwT[0:1, :]
                pyi = pwT[1:2, :]
                pzi = pwT[2:3, :]
                bic = jnp.transpose(b_ref[pl.ds(i0c, WI), :])
                ig2 = i0c + jax.lax.broadcasted_iota(jnp.int32, (JC, WI), 1)
                sqi = pxi * pxi + pyi * pyi + pzi * pzi
                d2q = sqj + sqi - 2.0 * (pxj * pxi + pyj * pyi + pzj * pzi)
                mask = ((d2q <= CUTOFF * CUTOFF) & (bjc == bic)
                        & (jg2 != ig2) & (ig2 >= i0o))
                dx = pxj - pxi
                dy = pyj - pyi
                dz = pzj - pzi
                distd = jnp.sqrt(dx * dx + dy * dy + dz * dz)
                ccd = 0.5 * (jnp.cos(distd * jnp.pi / CUTOFF) + 1.0)
                pmd = jnp.where(mask, ccd, 0.0)
                pmblk = jnp.where(selb,
                                  jnp.concatenate([pmd] * JC, axis=1), 0.0)
                dx3 = pxj3 - pw[:, 0:1].reshape(1, WI, 1)
                dy3 = pyj3 - pw[:, 1:2].reshape(1, WI, 1)
                dz3 = pzj3 - pw[:, 2:3].reshape(1, WI, 1)
                dist3 = jnp.sqrt(dx3 * dx3 + dy3 * dy3 + dz3 * dz3)
                dflat = dist3.reshape(JC * WI, 1)
                demb = jnp.exp(coeff * (dflat - offs) ** 2)
                aa = _ssp(_dotd(demb, m1t) + m1b)
                w = _dotd(aa, m2t) + m2b
                vlw = vl_scr[pl.ds(i0c, WI), :].reshape(1, WI, HID)
                e0 = (w.reshape(JC, WI, HID) * vlw).reshape(JC * WI, HID)
                agg_scr[pl.ds(c * JC, JC), :] += _dot(pmblk, e0)
                return carry2

            jax.lax.fori_loop(0, wn_ref[m, cj], win_body, 0)
            return carry

        jax.lax.fori_loop(0, nch_tile, chunk_body, 0)

        agg = agg_scr[...]
        h = _ssp(_dotd(agg, v1t_ref[...]) + v1b_ref[...])
        upd = _dotd(h, v2t_ref[...]) + v2b_ref[...]
        v_scr[wp, pl.ds(j0, TJ), :] = v_scr[rp, pl.ds(j0, TJ), :] + upd

    @pl.when(ph == NUM_LAYERS + 1)
    def _readout():
        rp = NUM_LAYERS % 2
        v = v_scr[rp, pl.ds(j0, TJ), :]
        h = _ssp(_dotd(v, u1t_ref[...]) + u1b_ref[...])
        out_ref[...] = _dotd(h, u2t_ref[...]) + u2b_ref[...]


def _cnn_body(x0_ref, w1_ref, b1_ref, w2_ref, b2_ref,
              w31_ref, b31_ref, w32_ref, b32_ref,
              w4_ref, b4_ref, fc1_ref, fc1b_ref, fc2_ref, fc2b_ref,
              out_ref):
    rows = x0_ref.shape[0]
    rh = 32
    hidx = jax.lax.rem(jax.lax.broadcasted_iota(jnp.int32, (rows, 1), 0), rh)

    def conv(x, w_ref, b_ref):
        c = x.shape[1]
        zr = jnp.zeros((1, c), _F32)
        xm = jnp.concatenate([zr, x[:-1, :]], axis=0)
        xm = jnp.where(hidx == 0, 0.0, xm)
        xp = jnp.concatenate([x[1:, :], zr], axis=0)
        xp = jnp.where(hidx == rh - 1, 0.0, xp)
        return (_dotd(xm, w_ref[0]) + _dotd(x, w_ref[1]) + _dotd(xp, w_ref[2])
                + b_ref[...])

    x = _lrelu(conv(x0_ref[...], w1_ref, b1_ref))
    x = _lrelu(conv(x, w2_ref, b2_ref))
    res = x
    x = _lrelu(conv(x, w31_ref, b31_ref))
    x = _lrelu(conv(x, w32_ref, b32_ref))
    x = res + x
    x = _lrelu(conv(x, w4_ref, b4_ref))
    x3 = x.reshape(rows // rh, rh, 256)
    acc = jnp.zeros((rows // rh, 64), _F32)
    for h in range(rh):
        acc = acc + _dotd(x3[:, h, :], fc1_ref[h])
    acc = _lrelu(acc + fc1b_ref[...])
    out_ref[...] = _dotd(acc, fc2_ref[...]) + fc2b_ref[...]


def _sc_segsum(u, bidx, nseg):
    n = u.shape[1]
    ns = 16
    rows = n // ns
    mesh = plsc.VectorSubcoreMesh(core_axis_name="c", subcore_axis_name="s")
    zeros = jnp.zeros((nseg, u.shape[2]), _F32)

    @functools.partial(
        pl.kernel, mesh=mesh,
        out_type=jax.ShapeDtypeStruct((2, nseg, u.shape[2]), _F32),
        scratch_types=[
            pltpu.VMEM((rows,), jnp.int32),
            pltpu.VMEM((rows, u.shape[2]), _F32),
            pltpu.VMEM_SHARED((nseg, u.shape[2]), _F32),
        ],
    )
    def k(u_hbm, b_hbm, z_hbm, out_hbm, idx_v, rows_v, shared):
        c = jax.lax.axis_index("c")
        s = jax.lax.axis_index("s")
        base = s * rows

        @pl.when(s == 0)
        def _():
            pltpu.sync_copy(z_hbm, shared)

        plsc.subcore_barrier()
        pltpu.sync_copy(b_hbm.at[c, pl.ds(base, rows)], idx_v)
        pltpu.sync_copy(u_hbm.at[c, pl.ds(base, rows)], rows_v)
        for t in range(ns):
            @pl.when(s == t)
            def _():
                pltpu.sync_copy(rows_v, shared.at[idx_v], add=True)

            plsc.subcore_barrier()

        @pl.when(s == 0)
        def _():
            pltpu.sync_copy(shared, out_hbm.at[c])

    return k(u, bidx, zeros)


def _fc_body(seg_ref, fw1t_ref, fb1_ref, fa_ref, fw2t_ref, fb2_ref, out_ref):
    for mm in range(2):
        x = seg_ref[mm]
        h = _dotd(x, fw1t_ref[mm]) + fb1_ref[mm]
        h = jnp.where(h >= 0, h, fa_ref[mm] * h)
        out_ref[mm] = _dotd(h, fw2t_ref[mm]) + fb2_ref[mm]


def _chunk_windows(batch):
    nch = batch.shape[0] // JC
    cidx = jnp.arange(nch) * JC
    lo = jnp.searchsorted(batch, batch[cidx], side="left")
    hi = jnp.searchsorted(batch, batch[cidx + (JC - 1)], side="right")
    lo8 = ((lo // 8) * 8).astype(jnp.int32)
    wn = ((hi - lo8 + (WI - 1)) // WI).astype(jnp.int32)
    return lo8, wn


def _stack_schnet(params):
    ms = [params["m1"], params["m2"]]
    out = {}
    out["emb"] = jnp.stack([p["emb"] for p in ms])
    for nm, src in [("m1t", "mlp1_w"), ("m2t", "mlp2_w"), ("lint", "lin_w"),
                    ("v1t", "v1_w"), ("v2t", "v2_w")]:
        out[nm] = jnp.stack([
            jnp.stack([lp[src].T for lp in p["layers"]]) for p in ms])
    for nm, src in [("m1b", "mlp1_b"), ("m2b", "mlp2_b"),
                    ("v1b", "v1_b"), ("v2b", "v2_b")]:
        out[nm] = jnp.stack([
            jnp.stack([lp[src][None, :] for lp in p["layers"]]) for p in ms])
    out["u1t"] = jnp.stack([p["u1_w"].T for p in ms])
    out["u1b"] = jnp.stack([p["u1_b"][None, :] for p in ms])
    out["u2t"] = jnp.stack([p["u2_w"].T for p in ms])
    out["u2b"] = jnp.stack([p["u2_b"][None, :] for p in ms])
    fcs = [params["fc1"], params["fc2"]]
    out["fw1t"] = jnp.stack([p["w1"].T for p in fcs])
    out["fb1"] = jnp.stack([p["b1"][None, :] for p in fcs])
    out["fa"] = jnp.stack([p["a"][None, :] for p in fcs])
    out["fw2t"] = jnp.stack([p["w2"].T for p in fcs])
    out["fb2"] = jnp.stack([p["b2"][None, :] for p in fcs])
    for nm in ("m1t", "m2t", "lint", "v1t", "v2t", "u1t", "u2t",
               "fw1t", "fw2t"):
        out[nm] = out[nm].astype(jnp.bfloat16)
    return out


@jax.jit
def kernel(pos1, z1, pos1_batch, pos2, z2, pos2_batch, params):
    n = pos1.shape[0]
    ntj = n // TJ
    g = NUM_GRAPHS

    sp = _stack_schnet(params)
    pos = jnp.stack([pos1, pos2])
    z = jnp.stack([z1, z2]).astype(jnp.int32)[:, :, None]
    b1 = pos1_batch.astype(jnp.int32)
    b2 = pos2_batch.astype(jnp.int32)
    bat = jnp.stack([b1, b2])[:, :, None]
    batT = jnp.stack([b1, b2])[:, None, :]
    offs = jnp.linspace(0.0, CUTOFF, NG)
    coeff = (-0.5 / (offs[1] - offs[0]) ** 2)[None, None]
    offs = offs[None, :]

    ilo1, wn1 = _chunk_windows(b1)
    ilo2, wn2 = _chunk_windows(b2)
    ilo = jnp.stack([ilo1, ilo2])
    wn = jnp.stack([wn1, wn2])

    def full(shape):
        nd = len(shape)
        return pl.BlockSpec(shape, lambda m, ph, bj, *_: (0,) * nd)

    def per_m(shape):
        nd = len(shape)
        return pl.BlockSpec((None,) + shape,
                            lambda m, ph, bj, *_: (m,) + (0,) * nd)

    def per_ml(shape):
        nd = len(shape)

        def imap(m, ph, bj, *_):
            l = jnp.clip(ph - 1, 0, NUM_LAYERS - 1)
            return (m, l) + (0,) * nd

        return pl.BlockSpec((None, None) + shape, imap)

    grid_spec = pltpu.PrefetchScalarGridSpec(
        num_scalar_prefetch=2,
        grid=(2, NUM_LAYERS + 2, ntj),
        in_specs=[
            per_m((n, 3)),
            per_m((n, 1)),
            per_m((n, 1)),
            per_m((1, n)),
            per_m((100, HID)),
            per_ml((NG, HID)),
            per_ml((1, HID)),
            per_ml((HID, HID)),
            per_ml((1, HID)),
            per_ml((HID, HID)),
            per_ml((HID, HID)),
            per_ml((1, HID)),
            per_ml((HID, HID)),
            per_ml((1, HID)),
            per_m((HID, HID // 2)),
            per_m((1, HID // 2)),
            per_m((HID // 2, OUTC)),
            per_m((1, OUTC)),
            full((1, NG)),
            full((1, 1)),
        ],
        out_specs=pl.BlockSpec((None, TJ, OUTC),
                               lambda m, ph, bj, *_: (m, bj, 0)),
        scratch_shapes=[
            pltpu.VMEM((2, n, HID), _F32),
            pltpu.VMEM((n, HID), _F32),
            pltpu.VMEM((TJ, HID), _F32),
        ],
    )

    u = pl.pallas_call(
        _schnet_body,
        grid_spec=grid_spec,
        out_shape=jax.ShapeDtypeStruct((2, n, OUTC), _F32),
    )(ilo, wn,
      pos, z, bat, batT, sp["emb"],
      sp["m1t"], sp["m1b"], sp["m2t"], sp["m2b"], sp["lint"],
      sp["v1t"], sp["v1b"], sp["v2t"], sp["v2b"],
      sp["u1t"], sp["u1b"], sp["u2t"], sp["u2b"],
      offs, coeff)

    seg = _sc_segsum(u, jnp.stack([b1, b2]), g)

    pred = pl.pallas_call(
        _fc_body,
        out_shape=jax.ShapeDtypeStruct((2, g, OUTC), _F32),
    )(seg, sp["fw1t"], sp["fb1"], sp["fa"], sp["fw2t"], sp["fb2"])

    x0 = jnp.concatenate([pred[0].reshape(-1, 1), pred[1].reshape(-1, 1)],
                         axis=1)

    cp = params["cnn"]
    bf = jnp.bfloat16
    w1 = jnp.transpose(cp["c1_w"], (2, 1, 0)).astype(bf)
    w2 = jnp.transpose(cp["c2_w"], (2, 1, 0)).astype(bf)
    w31 = jnp.transpose(cp["c31_w"], (2, 1, 0)).astype(bf)
    w32 = jnp.transpose(cp["c32_w"], (2, 1, 0)).astype(bf)
    w4 = jnp.transpose(cp["c4_w"], (2, 1, 0)).astype(bf)
    fc1 = jnp.transpose(cp["fc1_w"].reshape(64, 256, 32), (2, 1, 0)).astype(bf)
    out = pl.pallas_call(
        _cnn_body,
        out_shape=jax.ShapeDtypeStruct((g, NUM_CLASS), _F32),
    )(x0, w1, cp["c1_b"][None, :], w2, cp["c2_b"][None, :],
      w31, cp["c31_b"][None, :], w32, cp["c32_b"][None, :],
      w4, cp["c4_b"][None, :], fc1, cp["fc1_b"][None, :],
      cp["fc2_w"].T.astype(bf), cp["fc2_b"][None, :])
    return out
